# baseline (speedup 1.0000x reference)
import functools

import jax
import jax.numpy as jnp
from jax import lax
from jax.experimental import pallas as pl
from jax.experimental.pallas import tpu as pltpu
from jax.experimental.pallas import tpu_sc as plsc

N = 10000
E = 320000
D = 128

NC = 2
NS = 16
NW = NC * NS
G = 128

NBUF = 2
EPW = -(-E // NW)
NCH = -(-(-(-EPW // G)) // NBUF) * NBUF
E_PAD = NW * NCH * G
N_PAD = 10240
RPS = N_PAD // NS
NCH2 = NCH // 2


def _sc_agg_build(want_deg):
    mesh = plsc.VectorSubcoreMesh(
        core_axis_name="c", subcore_axis_name="s", num_cores=NC, num_subcores=NS
    )
    out_type = [jax.ShapeDtypeStruct((NC, N_PAD, D), jnp.float32)]
    scratch = [
        pltpu.VMEM((NCH2, G), jnp.int32),
        pltpu.VMEM((NCH2, G), jnp.int32),
        pltpu.VMEM((NBUF, G, D), jnp.float32),
        pltpu.VMEM_SHARED((N_PAD, D), jnp.float32),
        [pltpu.SemaphoreType.DMA] * NBUF,
    ]
    if want_deg:
        out_type.append(jax.ShapeDtypeStruct((NC, N_PAD), jnp.float32))
        scratch += [
            pltpu.VMEM((G,), jnp.float32),
            pltpu.VMEM_SHARED((N_PAD,), jnp.float32),
        ]

    def body(y_hbm, src_hbm, dst_hbm, zrows_hbm, zvec_hbm, *refs):
        if want_deg:
            agg_out, deg_out, src_v, dst_v, rows_v, agg_sh, sems, ones_v, deg_sh = refs
        else:
            agg_out, src_v, dst_v, rows_v, agg_sh, sems = refs
        c = lax.axis_index("c")
        s = lax.axis_index("s")
        wid = c * NS + s

        pltpu.sync_copy(zrows_hbm, agg_sh.at[pl.ds(s * RPS, RPS)])
        if want_deg:
            pltpu.sync_copy(zvec_hbm, deg_sh.at[pl.ds(s * RPS, RPS)])
            for k in range(G // 16):
                ones_v[pl.ds(k * 16, 16)] = jnp.ones((16,), jnp.float32)
        plsc.subcore_barrier()

        def start_gather(j, t):
            pltpu.async_copy(y_hbm.at[src_v.at[j]], rows_v.at[t], sems[t])

        def wait_gather(j, t):
            pltpu.make_async_copy(y_hbm.at[src_v.at[j]], rows_v.at[t], sems[t]).wait()

        for h in range(2):
            pltpu.sync_copy(src_hbm.at[wid, pl.ds(h * NCH2, NCH2)], src_v)
            pltpu.sync_copy(dst_hbm.at[wid, pl.ds(h * NCH2, NCH2)], dst_v)

            Q = NCH2 // NBUF
            for t in range(NBUF):
                start_gather(t * Q, t)

            def chunk(jj, carry):
                for t in range(NBUF):
                    j = t * Q + jj
                    wait_gather(j, t)
                    pltpu.sync_copy(rows_v.at[t], agg_sh.at[dst_v.at[j]], add=True)
                    if want_deg:
                        pltpu.sync_copy(ones_v, deg_sh.at[dst_v.at[j]], add=True)

                    @pl.when(jj + 1 < Q)
                    def _():
                        start_gather(j + 1, t)

                return carry

            lax.fori_loop(0, Q, chunk, 0)

        plsc.subcore_barrier()

        pltpu.sync_copy(
            agg_sh.at[pl.ds(s * RPS, RPS)], agg_out.at[c, pl.ds(s * RPS, RPS)]
        )
        if want_deg:
            pltpu.sync_copy(
                deg_sh.at[pl.ds(s * RPS, RPS)], deg_out.at[c, pl.ds(s * RPS, RPS)]
            )

    return pl.kernel(body, out_type=out_type, mesh=mesh, scratch_types=scratch)


_sc_agg_deg = _sc_agg_build(True)
_sc_agg = _sc_agg_build(False)


BN = 1000
_GRID = N // BN


def _mm_body(x_ref, w_ref, o_ref):
    o_ref[...] = lax.dot_general(
        x_ref[...], w_ref[...], (((1,), (1,)), ((), ())),
        preferred_element_type=jnp.float32,
    )


_mm = pl.pallas_call(
    _mm_body,
    grid=(_GRID,),
    in_specs=[
        pl.BlockSpec((BN, D), lambda i: (i, 0)),
        pl.BlockSpec((D, D), lambda i: (0, 0)),
    ],
    out_specs=pl.BlockSpec((BN, D), lambda i: (i, 0)),
    out_shape=jax.ShapeDtypeStruct((N, D), jnp.float32),
)


def _combine_mm_body(p_ref, y_ref, dg_ref, b_ref, w_ref, o_ref):
    agg = p_ref[0] + p_ref[1]
    deg = dg_ref[0] + dg_ref[1] + 1.0
    h = (agg + y_ref[...]) / deg + b_ref[...]
    h = jnp.where(h >= 0.0, h, 0.01 * h)
    o_ref[...] = lax.dot_general(
        h, w_ref[...], (((1,), (1,)), ((), ())),
        preferred_element_type=jnp.float32,
    )


_combine_mm = pl.pallas_call(
    _combine_mm_body,
    grid=(_GRID,),
    in_specs=[
        pl.BlockSpec((NC, BN, D), lambda i: (0, i, 0)),
        pl.BlockSpec((BN, D), lambda i: (i, 0)),
        pl.BlockSpec((NC, BN, 1), lambda i: (0, i, 0)),
        pl.BlockSpec((1, D), lambda i: (0, 0)),
        pl.BlockSpec((D, D), lambda i: (0, 0)),
    ],
    out_specs=pl.BlockSpec((BN, D), lambda i: (i, 0)),
    out_shape=jax.ShapeDtypeStruct((N, D), jnp.float32),
)


def _combine_body(p_ref, y_ref, dg_ref, b_ref, o_ref):
    agg = p_ref[0] + p_ref[1]
    deg = dg_ref[0] + dg_ref[1] + 1.0
    h = (agg + y_ref[...]) / deg + b_ref[...]
    o_ref[...] = jnp.where(h >= 0.0, h, 0.01 * h)


_combine = pl.pallas_call(
    _combine_body,
    grid=(_GRID,),
    in_specs=[
        pl.BlockSpec((NC, BN, D), lambda i: (0, i, 0)),
        pl.BlockSpec((BN, D), lambda i: (i, 0)),
        pl.BlockSpec((NC, BN, 1), lambda i: (0, i, 0)),
        pl.BlockSpec((1, D), lambda i: (0, 0)),
    ],
    out_specs=pl.BlockSpec((BN, D), lambda i: (i, 0)),
    out_shape=jax.ShapeDtypeStruct((N, D), jnp.float32),
)


def kernel(feat, edge_index, W1, b1, W2, b2):
    ei = jnp.asarray(edge_index, jnp.int32)
    pad = E_PAD - E
    src = jnp.concatenate([ei[0], jnp.zeros((pad,), jnp.int32)]).reshape(NW, NCH, G)
    dst = jnp.concatenate([ei[1], jnp.full((pad,), N, jnp.int32)]).reshape(NW, NCH, G)
    zrows = jnp.zeros((RPS, D), jnp.float32)
    zvec = jnp.zeros((RPS,), jnp.float32)
    b1r = b1.reshape(1, D)
    b2r = b2.reshape(1, D)

    y1 = _mm(feat, W1)
    p1, dg = _sc_agg_deg(y1, src, dst, zrows, zvec)
    dg3 = dg.reshape(NC, N_PAD, 1)
    y2 = _combine_mm(p1, y1, dg3, b1r, W2)
    p2 = _sc_agg(y2, src, dst, zrows, zvec)
    if isinstance(p2, (tuple, list)):
        p2 = p2[0]
    out = _combine(p2, y2, dg3, b2r)
    return out


# speedup vs baseline: 1.0002x; 1.0002x over previous
import functools

import jax
import jax.numpy as jnp
from jax import lax
from jax.experimental import pallas as pl
from jax.experimental.pallas import tpu as pltpu
from jax.experimental.pallas import tpu_sc as plsc

N = 10000
E = 320000
D = 128

NC = 2
NS = 16
NW = NC * NS
G = 128

NBUF = 2
EPW = -(-E // NW)
NCH = -(-(-(-EPW // G)) // NBUF) * NBUF
E_PAD = NW * NCH * G
N_PAD = 10240
RPS = N_PAD // NS
NCH2 = NCH // 2
Q2 = NCH2 // NBUF
GB = G * D * 4
DB = G * 4


def _sc_agg_build(want_deg):
    mesh = plsc.VectorSubcoreMesh(
        core_axis_name="c", subcore_axis_name="s", num_cores=NC, num_subcores=NS
    )
    out_type = [jax.ShapeDtypeStruct((NC, N_PAD, D), jnp.float32)]
    scratch = [
        pltpu.VMEM((NCH2, G), jnp.int32),
        pltpu.VMEM((NCH2, G), jnp.int32),
        pltpu.VMEM((NBUF, G, D), jnp.float32),
        pltpu.VMEM_SHARED((N_PAD, D), jnp.float32),
        [pltpu.SemaphoreType.DMA] * NBUF,
        [pltpu.SemaphoreType.DMA] * NBUF,
    ]
    if want_deg:
        out_type.append(jax.ShapeDtypeStruct((NC, N_PAD), jnp.float32))
        scratch += [
            pltpu.VMEM((G,), jnp.float32),
            pltpu.VMEM_SHARED((N_PAD,), jnp.float32),
            pltpu.SemaphoreType.DMA,
        ]

    def body(y_hbm, src_hbm, dst_hbm, zrows_hbm, zvec_hbm, *refs):
        if want_deg:
            (agg_out, deg_out, src_v, dst_v, rows_v, agg_sh, gsems, ssems,
             ones_v, deg_sh, dsem) = refs
        else:
            agg_out, src_v, dst_v, rows_v, agg_sh, gsems, ssems = refs
        c = lax.axis_index("c")
        s = lax.axis_index("s")
        wid = c * NS + s

        pltpu.sync_copy(zrows_hbm, agg_sh.at[pl.ds(s * RPS, RPS)])
        if want_deg:
            pltpu.sync_copy(zvec_hbm, deg_sh.at[pl.ds(s * RPS, RPS)])
            for k in range(G // 16):
                ones_v[pl.ds(k * 16, 16)] = jnp.ones((16,), jnp.float32)
        plsc.subcore_barrier()

        for h in range(2):
            pltpu.sync_copy(src_hbm.at[wid, pl.ds(h * NCH2, NCH2)], src_v)
            pltpu.sync_copy(dst_hbm.at[wid, pl.ds(h * NCH2, NCH2)], dst_v)

            for t in range(NBUF):
                pltpu.async_copy(y_hbm.at[src_v.at[t * Q2]], rows_v.at[t], gsems[t])

            def chunk(jj, carry):
                for t in range(NBUF):
                    j = t * Q2 + jj
                    pltpu.make_async_copy(
                        zrows_hbm.at[pl.ds(0, G)], rows_v.at[t], gsems[t]
                    ).wait()
                    pltpu.async_copy(
                        rows_v.at[t], agg_sh.at[dst_v.at[j]], ssems[t], add=True
                    )
                    if want_deg:
                        pltpu.async_copy(ones_v, deg_sh.at[dst_v.at[j]], dsem, add=True)
                    pltpu.make_async_copy(
                        zrows_hbm.at[pl.ds(0, G)], rows_v.at[t], ssems[t]
                    ).wait()
                    if want_deg:
                        pltpu.make_async_copy(
                            zvec_hbm.at[pl.ds(0, G)], ones_v, dsem
                        ).wait()

                    @pl.when(jj + 1 < Q2)
                    def _():
                        pltpu.async_copy(
                            y_hbm.at[src_v.at[j + 1]], rows_v.at[t], gsems[t]
                        )

                return carry

            lax.fori_loop(0, Q2, chunk, 0)

        plsc.subcore_barrier()

        pltpu.sync_copy(
            agg_sh.at[pl.ds(s * RPS, RPS)], agg_out.at[c, pl.ds(s * RPS, RPS)]
        )
        if want_deg:
            pltpu.sync_copy(
                deg_sh.at[pl.ds(s * RPS, RPS)], deg_out.at[c, pl.ds(s * RPS, RPS)]
            )

    return pl.kernel(body, out_type=out_type, mesh=mesh, scratch_types=scratch)


_sc_agg_deg = _sc_agg_build(True)
_sc_agg = _sc_agg_build(False)


BN = 1000
_GRID = N // BN


def _mm_body(x_ref, w_ref, o_ref):
    o_ref[...] = lax.dot_general(
        x_ref[...], w_ref[...], (((1,), (1,)), ((), ())),
        preferred_element_type=jnp.float32,
    )


_mm = pl.pallas_call(
    _mm_body,
    grid=(_GRID,),
    in_specs=[
        pl.BlockSpec((BN, D), lambda i: (i, 0)),
        pl.BlockSpec((D, D), lambda i: (0, 0)),
    ],
    out_specs=pl.BlockSpec((BN, D), lambda i: (i, 0)),
    out_shape=jax.ShapeDtypeStruct((N, D), jnp.float32),
)


def _combine_mm_body(p_ref, y_ref, dg_ref, b_ref, w_ref, o_ref):
    agg = p_ref[0] + p_ref[1]
    deg = dg_ref[0] + dg_ref[1] + 1.0
    h = (agg + y_ref[...]) / deg + b_ref[...]
    h = jnp.where(h >= 0.0, h, 0.01 * h)
    o_ref[...] = lax.dot_general(
        h, w_ref[...], (((1,), (1,)), ((), ())),
        preferred_element_type=jnp.float32,
    )


_combine_mm = pl.pallas_call(
    _combine_mm_body,
    grid=(_GRID,),
    in_specs=[
        pl.BlockSpec((NC, BN, D), lambda i: (0, i, 0)),
        pl.BlockSpec((BN, D), lambda i: (i, 0)),
        pl.BlockSpec((NC, BN, 1), lambda i: (0, i, 0)),
        pl.BlockSpec((1, D), lambda i: (0, 0)),
        pl.BlockSpec((D, D), lambda i: (0, 0)),
    ],
    out_specs=pl.BlockSpec((BN, D), lambda i: (i, 0)),
    out_shape=jax.ShapeDtypeStruct((N, D), jnp.float32),
)


def _combine_body(p_ref, y_ref, dg_ref, b_ref, o_ref):
    agg = p_ref[0] + p_ref[1]
    deg = dg_ref[0] + dg_ref[1] + 1.0
    h = (agg + y_ref[...]) / deg + b_ref[...]
    o_ref[...] = jnp.where(h >= 0.0, h, 0.01 * h)


_combine = pl.pallas_call(
    _combine_body,
    grid=(_GRID,),
    in_specs=[
        pl.BlockSpec((NC, BN, D), lambda i: (0, i, 0)),
        pl.BlockSpec((BN, D), lambda i: (i, 0)),
        pl.BlockSpec((NC, BN, 1), lambda i: (0, i, 0)),
        pl.BlockSpec((1, D), lambda i: (0, 0)),
    ],
    out_specs=pl.BlockSpec((BN, D), lambda i: (i, 0)),
    out_shape=jax.ShapeDtypeStruct((N, D), jnp.float32),
)


def kernel(feat, edge_index, W1, b1, W2, b2):
    ei = jnp.asarray(edge_index, jnp.int32)
    pad = E_PAD - E
    src = jnp.concatenate([ei[0], jnp.zeros((pad,), jnp.int32)]).reshape(NW, NCH, G)
    dst = jnp.concatenate([ei[1], jnp.full((pad,), N, jnp.int32)]).reshape(NW, NCH, G)
    zrows = jnp.zeros((RPS, D), jnp.float32)
    zvec = jnp.zeros((RPS,), jnp.float32)
    b1r = b1.reshape(1, D)
    b2r = b2.reshape(1, D)

    y1 = _mm(feat, W1)
    p1, dg = _sc_agg_deg(y1, src, dst, zrows, zvec)
    dg3 = dg.reshape(NC, N_PAD, 1)
    y2 = _combine_mm(p1, y1, dg3, b1r, W2)
    p2 = _sc_agg(y2, src, dst, zrows, zvec)
    if isinstance(p2, (tuple, list)):
        p2 = p2[0]
    out = _combine(p2, y2, dg3, b2r)
    return out
